# baseline (speedup 1.0000x reference)
import functools

import jax
import jax.numpy as jnp
from jax import lax
from jax.experimental import pallas as pl
from jax.experimental.pallas import tpu as pltpu
from jax.experimental.pallas import tpu_sc as plsc

B = 4096
HIST = 200
EMB = 16
NC = 2
NS = 16
NW = NC * NS
BPW = B // NW
C = 16
NCH = BPW // C
ROWS = C * HIST
IPD = 128
DPC = ROWS // IPD


def _scores_sc(ui_idx, item_idx, inum, src_tab, dst_tab):
    mesh = plsc.VectorSubcoreMesh(
        core_axis_name="c", subcore_axis_name="s", num_cores=NC, num_subcores=NS
    )

    @functools.partial(
        pl.kernel,
        out_type=jax.ShapeDtypeStruct((B,), jnp.float32),
        mesh=mesh,
        compiler_params=pltpu.CompilerParams(
            needs_layout_passes=False, use_tc_tiling_on_sc=False
        ),
        scratch_types=[
            pltpu.VMEM((BPW * HIST // IPD, IPD), jnp.int32),
            pltpu.VMEM((ROWS, EMB), jnp.float32),
            pltpu.VMEM((BPW,), jnp.int32),
            pltpu.VMEM((BPW, EMB), jnp.float32),
            pltpu.VMEM((BPW,), jnp.float32),
            pltpu.VMEM((C, EMB), jnp.float32),
            pltpu.VMEM((BPW,), jnp.float32),
            pltpu.SemaphoreType.DMA,
        ],
    )
    def k(ui_hbm, item_hbm, inum_hbm, src_hbm, dst_hbm, out_hbm,
          idx_v, rows_v, tidx_v, tgt_v, inum_v, dots_v, res_v, sem):
        wid = lax.axis_index("s") * NC + lax.axis_index("c")
        wrows = BPW * HIST // IPD

        pltpu.sync_copy(ui_hbm.at[pl.ds(wid * wrows, wrows)], idx_v)
        pltpu.sync_copy(item_hbm.at[pl.ds(wid * BPW, BPW)], tidx_v)
        pltpu.sync_copy(inum_hbm.at[pl.ds(wid * BPW, BPW)], inum_v)
        pltpu.async_copy(dst_hbm.at[tidx_v], tgt_v, sem).wait()

        for c in range(NCH):
            cps = [
                pltpu.async_copy(
                    src_hbm.at[idx_v.at[c * DPC + j]],
                    rows_v.at[pl.ds(j * IPD, IPD)],
                    sem,
                )
                for j in range(DPC)
            ]
            for cp in cps:
                cp.wait()

            lanes = lax.iota(jnp.int32, 16)

            def row_body(i, _):
                base = i * HIST

                def acc_body(l, acc):
                    return acc + rows_v[base + l]

                acc = lax.fori_loop(
                    0, HIST, acc_body, jnp.zeros((EMB,), jnp.float32),
                    unroll=8,
                )
                prod = acc * tgt_v[c * C + i]
                col = jnp.full((16,), 0, jnp.int32) + i
                plsc.store_scatter(dots_v, [lanes, col], prod)
                return 0

            lax.fori_loop(0, C, row_body, 0)

            dots = jnp.zeros((16,), jnp.float32)
            for r in range(EMB):
                dots = dots + dots_v[r]

            x = inum_v[pl.ds(c * C, C)]
            yi = jnp.int32(0x5F3759DF) - (plsc.bitcast(x, jnp.int32) >> 1)
            y = plsc.bitcast(yi, jnp.float32)
            for _ in range(3):
                y = y * (1.5 - 0.5 * x * y * y)

            s = dots * y
            res_v[pl.ds(c * C, C)] = 1.0 / (1.0 + jnp.exp(-s))

        pltpu.sync_copy(res_v, out_hbm.at[pl.ds(wid * BPW, BPW)])

    return k(ui_idx, item_idx, inum, src_tab, dst_tab)


def kernel(user_inter, item, item_num, user, item_src_emb, item_dst_emb,
           user_bias, item_bias):
    ui_idx = user_inter.astype(jnp.int32).reshape(B * HIST // IPD, IPD)
    item_idx = item.astype(jnp.int32).reshape(B)
    inum = item_num.astype(jnp.float32).reshape(B)
    scores = _scores_sc(ui_idx, item_idx, inum, item_src_emb, item_dst_emb)
    return scores.reshape(B, 1)
